# Initial kernel scaffold; baseline (speedup 1.0000x reference)
import jax
import jax.numpy as jnp
from jax.experimental import pallas as pl


def kernel(x_nchw, weight):
    raise NotImplementedError("write your pallas kernel here")


# speedup vs baseline: 1.7426x; 1.7426x over previous
import jax
import jax.numpy as jnp
from jax.experimental import pallas as pl
from jax.experimental.pallas import tpu as pltpu


def _se_fused_kernel(x_ref, w_ref, o_ref, *, inv_hw):
    x = x_ref[...]
    mean = jnp.sum(x, axis=-1, keepdims=True, dtype=jnp.float32) * inv_hw
    z = jnp.dot(w_ref[...], mean, preferred_element_type=jnp.float32)
    gate = jax.nn.sigmoid(z).astype(x.dtype)
    o_ref[...] = x * gate


def kernel(x_nchw, weight):
    N, C, H, W = x_nchw.shape
    HW = H * W
    x = x_nchw.reshape(N, C, HW)
    w = weight.astype(jnp.float32)

    import functools
    body = functools.partial(_se_fused_kernel, inv_hw=float(1.0 / HW))

    itemsize = jnp.dtype(x.dtype).itemsize
    cost = pl.CostEstimate(
        flops=3 * N * C * HW + 2 * N * C * C,
        transcendentals=N * C,
        bytes_accessed=2 * N * C * HW * itemsize + C * C * 4,
    )
    out = pl.pallas_call(
        body,
        out_shape=jax.ShapeDtypeStruct((N, C, HW), x.dtype),
        grid=(N,),
        in_specs=[
            pl.BlockSpec((pl.Squeezed(), C, HW), lambda n: (n, 0, 0)),
            pl.BlockSpec((C, C), lambda n: (0, 0)),
        ],
        out_specs=pl.BlockSpec((pl.Squeezed(), C, HW), lambda n: (n, 0, 0)),
        compiler_params=pltpu.CompilerParams(
            dimension_semantics=("parallel",),
            vmem_limit_bytes=64 * 1024 * 1024,
        ),
        cost_estimate=cost,
    )(x, w)
    return out.reshape(N, C, H, W)
